# baseline (speedup 1.0000x reference)
import math

import jax
import jax.numpy as jnp
from jax.experimental import pallas as pl
from jax.experimental.pallas import tpu as pltpu

_LM = [(l, m) for l in range(4) for m in range(l + 1)]
_NORM = [
    math.sqrt((2 * l + 1) / (4.0 * math.pi)
              * math.factorial(l - m) / math.factorial(l + m))
    for (l, m) in _LM
]


def _body(x_ref, o_ref):
    xv = x_ref[0]
    x0, x1, x2 = xv[0], xv[1], xv[2]

    xy = x0 * x0 + x1 * x1
    theta = jnp.arctan2(jnp.sqrt(xy) + 1e-5, x2 + 1e-5)
    phi = jnp.arctan2(x1 + 1e-5, x0 + 1e-5)

    c = jnp.cos(phi)
    s = jnp.sqrt(jnp.clip(1.0 - c * c, 1e-12, 1.0))

    P = {
        (0, 0): jnp.ones_like(c),
        (1, 0): c,
        (1, 1): -s,
        (2, 0): 0.5 * (3.0 * c * c - 1.0),
        (2, 1): -3.0 * c * s,
        (2, 2): 3.0 * (1.0 - c * c),
        (3, 0): 0.5 * c * (5.0 * c * c - 3.0),
        (3, 1): -1.5 * (5.0 * c * c - 1.0) * s,
        (3, 2): 15.0 * c * (1.0 - c * c),
        (3, 3): -15.0 * s * s * s,
    }
    cos_m = [jnp.ones_like(theta), jnp.cos(theta),
             jnp.cos(2.0 * theta), jnp.cos(3.0 * theta)]
    sin_m = [jnp.zeros_like(theta), jnp.sin(theta),
             jnp.sin(2.0 * theta), jnp.sin(3.0 * theta)]

    for ch, (l, m) in enumerate(_LM):
        amp = _NORM[ch] * P[(l, m)]
        ph = jnp.arctan2(amp * sin_m[m], amp * cos_m[m])
        o_ref[0, ch] = jax.lax.dot_general(
            ph, ph, (((0,), (0,)), ((), ())),
            preferred_element_type=jnp.float32)


def kernel(x):
    N, C, T, V = x.shape
    L = len(_LM)
    return pl.pallas_call(
        _body,
        grid=(N,),
        in_specs=[pl.BlockSpec((1, C, T, V), lambda n: (n, 0, 0, 0))],
        out_specs=pl.BlockSpec((1, L, V, V), lambda n: (n, 0, 0, 0)),
        out_shape=jax.ShapeDtypeStruct((N, L, V, V), x.dtype),
        compiler_params=pltpu.CompilerParams(
            dimension_semantics=("parallel",)),
    )(x)


# speedup vs baseline: 3.8434x; 3.8434x over previous
import math

import jax
import jax.numpy as jnp
from jax.experimental import pallas as pl
from jax.experimental.pallas import tpu as pltpu

_LM = [(l, m) for l in range(4) for m in range(l + 1)]
_NORM = [
    math.sqrt((2 * l + 1) / (4.0 * math.pi)
              * math.factorial(l - m) / math.factorial(l + m))
    for (l, m) in _LM
]


def _body(x_ref, o_ref):
    xv = x_ref[0]
    x0, x1, x2 = xv[0], xv[1], xv[2]

    xy = x0 * x0 + x1 * x1
    theta = jnp.arctan2(jnp.sqrt(xy) + 1e-5, x2 + 1e-5)
    phi = jnp.arctan2(x1 + 1e-5, x0 + 1e-5)

    c = jnp.cos(phi)
    s = jnp.sqrt(jnp.clip(1.0 - c * c, 1e-12, 1.0))

    P = {
        (0, 0): jnp.ones_like(c),
        (1, 0): c,
        (1, 1): -s,
        (2, 0): 0.5 * (3.0 * c * c - 1.0),
        (2, 1): -3.0 * c * s,
        (2, 2): 3.0 * (1.0 - c * c),
        (3, 0): 0.5 * c * (5.0 * c * c - 3.0),
        (3, 1): -1.5 * (5.0 * c * c - 1.0) * s,
        (3, 2): 15.0 * c * (1.0 - c * c),
        (3, 3): -15.0 * s * s * s,
    }
    cos_m = [jnp.ones_like(theta), jnp.cos(theta),
             jnp.cos(2.0 * theta), jnp.cos(3.0 * theta)]
    sin_m = [jnp.zeros_like(theta), jnp.sin(theta),
             jnp.sin(2.0 * theta), jnp.sin(3.0 * theta)]

    for ch, (l, m) in enumerate(_LM):
        amp = _NORM[ch] * P[(l, m)]
        ph = jnp.arctan2(amp * sin_m[m], amp * cos_m[m])
        o_ref[0, ch] = jax.lax.dot_general(
            ph, ph, (((1,), (1,)), ((), ())),
            preferred_element_type=jnp.float32)


def kernel(x):
    N, C, T, V = x.shape
    L = len(_LM)
    xt = jnp.swapaxes(x, 2, 3)
    return pl.pallas_call(
        _body,
        grid=(N,),
        in_specs=[pl.BlockSpec((1, C, V, T), lambda n: (n, 0, 0, 0))],
        out_specs=pl.BlockSpec((1, L, V, V), lambda n: (n, 0, 0, 0)),
        out_shape=jax.ShapeDtypeStruct((N, L, V, V), x.dtype),
        compiler_params=pltpu.CompilerParams(
            dimension_semantics=("parallel",)),
    )(xt)


# speedup vs baseline: 14.0026x; 3.6433x over previous
import math

import jax
import jax.numpy as jnp
import numpy as np
from jax.experimental import pallas as pl
from jax.experimental.pallas import tpu as pltpu

_PI = np.float32(math.pi)
_L = 10


def _body(x_ref, o_ref):
    xv = x_ref[0]
    x0, x1, x2 = xv[0], xv[1], xv[2]

    st = jnp.sqrt(x0 * x0 + x1 * x1) + 1e-5
    ct = x2 + 1e-5
    theta = jnp.arctan2(st, ct)

    b1 = theta
    t2 = 2.0 * theta
    b2 = jnp.where(t2 > _PI, t2 - 2.0 * _PI, t2)
    t3 = 3.0 * theta
    b3 = jnp.where(t3 > _PI, t3 - 2.0 * _PI, t3)

    sg2 = jnp.where(ct >= 0.0, _PI, -_PI)
    sg3 = jnp.where(3.0 * ct * ct >= st * st, _PI, -_PI)

    xa = x0 + 1e-5
    ya = x1 + 1e-5
    c = xa * jax.lax.rsqrt(jnp.maximum(xa * xa + ya * ya, 1e-36))
    c2 = c * c

    zero = jnp.zeros_like(c)
    neg_pi = jnp.full_like(c, -_PI)

    phases = [
        None,
        jnp.where(c < 0.0, neg_pi, zero),
        b1 - _PI,
        jnp.where(3.0 * c2 < 1.0, neg_pi, zero),
        jnp.where(c > 0.0, b1 - _PI, b1),
        b2,
        jnp.where(c * (5.0 * c2 - 3.0) < 0.0, neg_pi, zero),
        jnp.where(5.0 * c2 > 1.0, b1 - _PI, b1),
        jnp.where(c < 0.0, b2 - sg2, b2),
        b3 - sg3,
    ]

    o_ref[0, 0] = jnp.zeros_like(o_ref[0, 0])
    for ch in range(1, _L):
        ph = phases[ch]
        o_ref[0, ch] = jax.lax.dot_general(
            ph, ph, (((1,), (1,)), ((), ())),
            preferred_element_type=jnp.float32)


def kernel(x):
    N, C, T, V = x.shape
    xt = jnp.swapaxes(x, 2, 3)
    return pl.pallas_call(
        _body,
        grid=(N,),
        in_specs=[pl.BlockSpec((1, C, V, T), lambda n: (n, 0, 0, 0))],
        out_specs=pl.BlockSpec((1, _L, V, V), lambda n: (n, 0, 0, 0)),
        out_shape=jax.ShapeDtypeStruct((N, _L, V, V), x.dtype),
        compiler_params=pltpu.CompilerParams(
            dimension_semantics=("parallel",)),
    )(xt)
